# baseline (speedup 1.0000x reference)
import functools

import jax
import jax.numpy as jnp
from jax import lax
from jax.experimental import pallas as pl
from jax.experimental.pallas import tpu as pltpu
from jax.experimental.pallas import tpu_sc as plsc

_N = 10000
_E = 160000
_R = 8
_D = 256
_C = 16

_NC = 2
_NS = 16
_NW = _NC * _NS
_NP = 10240
_SEG_PAD = _NP * _R
_SPT = _SEG_PAD // _NW
_SB = 128
_NBLK = _SPT // _SB
_NBLK_TOT = _SEG_PAD // _SB
_GB = 128
_IC = 1024
_EPAD = _E + 1664

_BN = 256


def _sc_segsum_body(x_hbm, srcs_hbm, segs_hbm, bounds_hbm, zeros_hbm,
                    agg_hbm, cnt_hbm,
                    bvec, idx_c, seg_c, rows0, rows1, stage, cstage,
                    sem, gsem0, gsem1):
    wid = lax.axis_index("c") * _NS + lax.axis_index("s")
    pltpu.sync_copy(bounds_hbm.at[wid], bvec)
    lanes = lax.iota(jnp.int32, 16)
    ones = jnp.ones((16,), jnp.float32)
    lane0 = lanes == 0
    bv0 = bvec[pl.ds(0, 16)]
    bv1 = bvec[pl.ds(8, 16)]

    def sread(b):
        lo = jnp.sum(jnp.where(lanes == b, bv0, 0))
        hi = jnp.sum(jnp.where((lanes + 8 == b) & (b >= 16), bv1, 0))
        return lo + hi

    def block_body(b, _):
        seg_lo = pl.multiple_of(wid * _SPT + b * _SB, _SB)
        e_lo = sread(b)
        e_hi = sread(b + 1)
        e_start = jnp.bitwise_and(e_lo, jnp.int32(-8))
        nbatch = (e_hi - e_start + (_GB - 1)) // _GB
        nchunk = jnp.maximum((nbatch + 7) // 8, 1)

        dz = pltpu.async_copy(zeros_hbm, stage, sem)
        dzc = pltpu.async_copy(zeros_hbm.at[0, pl.ds(0, _SB)], cstage, sem)
        dz.wait()
        dzc.wait()

        def process(p, rows):
            def grp_body(g, _):
                seg16 = seg_c[pl.ds(p * _GB + g * 16, 16)]
                for k in range(16):
                    seg_sc = jnp.sum(jnp.where(lanes == k, seg16, 0))
                    loc = seg_sc - seg_lo
                    row = g * 16 + k

                    @pl.when((loc >= 0) & (loc < _SB))
                    def _():
                        for j in range(_D // 16):
                            sl = pl.ds(j * 16, 16)
                            stage[loc, sl] = stage[loc, sl] + rows[row, sl]
                        locv = jnp.full((16,), loc, jnp.int32)
                        plsc.addupdate_scatter(cstage, [locv], ones,
                                               mask=lane0)
                return 0

            lax.fori_loop(0, 8, grp_body, 0)

        def start_g(p, buf, gsem):
            off = pl.multiple_of(
                jnp.minimum(p * _GB, _IC - _GB), 8)
            pltpu.async_copy(x_hbm.at[idx_c.at[pl.ds(off, _GB)]], buf, gsem)

        def drain(buf, gsem):
            pltpu.make_async_copy(x_hbm.at[pl.ds(0, _GB)], buf, gsem).wait()

        def chunk_body(c, _):
            c0 = pl.multiple_of(e_start + c * _IC, 8)
            di = pltpu.async_copy(srcs_hbm.at[pl.ds(c0, _IC)], idx_c, sem)
            ds = pltpu.async_copy(segs_hbm.at[pl.ds(c0, _IC)], seg_c, sem)
            di.wait()
            ds.wait()
            nb_c = jnp.clip(nbatch - c * 8, 0, 8)
            npair = jnp.maximum((nb_c + 1) // 2, 1)

            start_g(0, rows0, gsem0)

            def pair_body(j, _):
                p0 = 2 * j
                start_g(p0 + 1, rows1, gsem1)
                drain(rows0, gsem0)
                process(p0, rows0)
                start_g(p0 + 2, rows0, gsem0)
                drain(rows1, gsem1)
                process(p0 + 1, rows1)
                return 0

            lax.fori_loop(0, npair, pair_body, 0)
            drain(rows0, gsem0)
            return 0

        lax.fori_loop(0, nchunk, chunk_body, 0)

        da = pltpu.async_copy(stage, agg_hbm.at[pl.ds(seg_lo, _SB)], sem)
        dc = pltpu.async_copy(cstage, cnt_hbm.at[pl.ds(seg_lo, _SB)], sem)
        da.wait()
        dc.wait()
        return 0

    lax.fori_loop(0, _NBLK, block_body, 0)


_sc_segsum = functools.partial(
    pl.kernel,
    _sc_segsum_body,
    out_type=(jax.ShapeDtypeStruct((_SEG_PAD, _D), jnp.float32),
              jax.ShapeDtypeStruct((_SEG_PAD,), jnp.float32)),
    scratch_types=[
        pltpu.VMEM((24,), jnp.int32),
        pltpu.VMEM((_IC,), jnp.int32),
        pltpu.VMEM((_IC,), jnp.int32),
        pltpu.VMEM((_GB, _D), jnp.float32),
        pltpu.VMEM((_GB, _D), jnp.float32),
        pltpu.VMEM((_SB, _D), jnp.float32),
        pltpu.VMEM((_SB,), jnp.float32),
        pltpu.SemaphoreType.DMA,
        pltpu.SemaphoreType.DMA,
        pltpu.SemaphoreType.DMA,
    ],
    compiler_params=pltpu.CompilerParams(needs_layout_passes=False),
)


def _tc_body(act, x_ref, a_ref, c_ref, root_ref, w_ref, b_ref, o_ref):
    inv = 1.0 / jnp.maximum(c_ref[...], 1.0)
    acc = jnp.dot(x_ref[...], root_ref[...],
                  preferred_element_type=jnp.float32)
    acc = acc + b_ref[...]
    for r in range(_R):
        a_r = a_ref[:, r * _D:(r + 1) * _D] * inv[:, r:r + 1]
        acc = acc + jnp.dot(a_r, w_ref[r],
                            preferred_element_type=jnp.float32)
    o_ref[...] = act(acc)


def _tc_layer(x, agg, cnt128, root, w, bias, act, dout):
    return pl.pallas_call(
        functools.partial(_tc_body, act),
        grid=(_NP // _BN,),
        in_specs=[
            pl.BlockSpec((_BN, _D), lambda i: (i, 0)),
            pl.BlockSpec((_BN, _R * _D), lambda i: (i, 0)),
            pl.BlockSpec((_BN, 128), lambda i: (i, 0)),
            pl.BlockSpec((_D, dout), lambda i: (0, 0)),
            pl.BlockSpec((_R, _D, dout), lambda i: (0, 0, 0)),
            pl.BlockSpec((1, dout), lambda i: (0, 0)),
        ],
        out_specs=pl.BlockSpec((_BN, dout), lambda i: (i, 0)),
        out_shape=jax.ShapeDtypeStruct((_NP, dout), jnp.float32),
    )(x, agg, cnt128, root, w, bias)


def _relu(v):
    return jnp.maximum(v, 0.0)


def _sigmoid(v):
    return 1.0 / (1.0 + jnp.exp(-v))


@jax.jit
def kernel(edge_index, edge_type, emb, W1, root1, b1, W2, root2, b2):
    src = edge_index[0]
    dst = edge_index[1]
    seg = dst * _R + edge_type
    seg_s, src_s = lax.sort((seg, src), num_keys=1)
    seg_p = jnp.concatenate(
        [seg_s, jnp.full((_EPAD - _E,), _SEG_PAD, jnp.int32)])
    src_p = jnp.concatenate(
        [src_s, jnp.zeros((_EPAD - _E,), jnp.int32)])

    bstart = jnp.searchsorted(
        seg_s,
        jnp.arange(_NBLK_TOT + 1, dtype=jnp.int32) * _SB).astype(jnp.int32)
    bt = bstart[jnp.arange(_NW)[:, None] * _NBLK
                + jnp.arange(_NBLK + 1)[None, :]]
    bt = jnp.pad(bt, ((0, 0), (0, 3)))

    zbuf = jnp.zeros((_SB, _D), jnp.float32)
    mesh = plsc.VectorSubcoreMesh(core_axis_name="c", subcore_axis_name="s",
                                  num_cores=_NC, num_subcores=_NS)
    sc_call = _sc_segsum(mesh=mesh)

    xp = jnp.pad(emb, ((0, _NP - _N), (0, 0)))
    agg1, cnt = sc_call(xp, src_p, seg_p, bt, zbuf)
    cnt128 = jnp.pad(cnt.reshape(_NP, _R), ((0, 0), (0, 128 - _R)))
    h1 = _tc_layer(xp, agg1.reshape(_NP, _R * _D), cnt128,
                   root1, W1, b1.reshape(1, _D), _relu, _D)

    agg2, _ = sc_call(h1, src_p, seg_p, bt, zbuf)
    w2p = jnp.pad(W2, ((0, 0), (0, 0), (0, 128 - _C)))
    root2p = jnp.pad(root2, ((0, 0), (0, 128 - _C)))
    b2p = jnp.pad(b2, (0, 128 - _C)).reshape(1, 128)
    out = _tc_layer(h1, agg2.reshape(_NP, _R * _D), cnt128,
                    root2p, w2p, b2p, _sigmoid, 128)
    return out[:_N, :_C]


# speedup vs baseline: 1.1890x; 1.1890x over previous
import functools

import jax
import jax.numpy as jnp
from jax import lax
from jax.experimental import pallas as pl
from jax.experimental.pallas import tpu as pltpu
from jax.experimental.pallas import tpu_sc as plsc

_N = 10000
_E = 160000
_R = 8
_D = 256
_C = 16

_NC = 2
_NS = 16
_NW = _NC * _NS
_NP = 10240
_SEG_PAD = _NP * _R
_SPT = _SEG_PAD // _NW
_SB = 128
_NBLK = _SPT // _SB
_NBLK_TOT = _SEG_PAD // _SB
_GB = 128
_EPAD = _E + 2 * _GB

_BN = 256


def _sc_segsum_body(x_hbm, srcs_hbm, segs_hbm, bounds_hbm, zeros_hbm,
                    agg_hbm, cnt_hbm,
                    bvec, idx_v, seg_v, rows_v, stage, cstage, sem):
    wid = lax.axis_index("c") * _NS + lax.axis_index("s")
    pltpu.sync_copy(bounds_hbm.at[wid], bvec)
    lanes = lax.iota(jnp.int32, 16)
    ones = jnp.ones((16,), jnp.float32)
    lane0 = lanes == 0
    bv0 = bvec[pl.ds(0, 16)]
    bv1 = bvec[pl.ds(8, 16)]

    def sread(b):
        lo = jnp.sum(jnp.where(lanes == b, bv0, 0))
        hi = jnp.sum(jnp.where((lanes + 8 == b) & (b >= 16), bv1, 0))
        return lo + hi

    def block_body(b, _):
        seg_lo = pl.multiple_of(wid * _SPT + b * _SB, _SB)
        e_lo = sread(b)
        e_hi = sread(b + 1)
        e_start = jnp.bitwise_and(e_lo, jnp.int32(-8))
        nbatch = (e_hi - e_start + (_GB - 1)) // _GB

        pltpu.sync_copy(zeros_hbm, stage)
        pltpu.sync_copy(zeros_hbm.at[pl.ds(0, _SB)], cstage)

        def batch_body(i, _):
            e0 = pl.multiple_of(e_start + i * _GB, 8)
            pltpu.sync_copy(srcs_hbm.at[pl.ds(e0, _GB)], idx_v)
            pltpu.sync_copy(segs_hbm.at[pl.ds(e0, _GB)], seg_v)
            pltpu.async_copy(x_hbm.at[idx_v], rows_v, sem).wait()

            def grp_body(g, _):
                seg16 = seg_v[pl.ds(g * 16, 16)]
                for k in range(16):
                    seg_sc = jnp.sum(jnp.where(lanes == k, seg16, 0))
                    seg_b = jnp.full((16,), seg_sc, jnp.int32)
                    in_blk = (seg_b >= seg_lo) & (seg_b < seg_lo + _SB)
                    base = (seg_b - seg_lo) * _D + lanes
                    row = g * 16 + k
                    for j in range(_D // 16):
                        val = rows_v[row, pl.ds(j * 16, 16)]
                        plsc.addupdate_scatter(stage, [base + j * 16], val,
                                               mask=in_blk)
                    plsc.addupdate_scatter(cstage, [seg_b - seg_lo], ones,
                                           mask=in_blk & lane0)
                return 0

            return lax.fori_loop(0, 8, grp_body, 0)

        lax.fori_loop(0, nbatch, batch_body, 0)
        pltpu.sync_copy(
            stage,
            agg_hbm.at[pl.ds(pl.multiple_of(seg_lo * _D, 8), _SB * _D)])
        pltpu.sync_copy(cstage, cnt_hbm.at[pl.ds(seg_lo, _SB)])
        return 0

    lax.fori_loop(0, _NBLK, block_body, 0)


_sc_segsum = functools.partial(
    pl.kernel,
    _sc_segsum_body,
    out_type=(jax.ShapeDtypeStruct((_SEG_PAD * _D,), jnp.float32),
              jax.ShapeDtypeStruct((_SEG_PAD,), jnp.float32)),
    scratch_types=[
        pltpu.VMEM((24,), jnp.int32),
        pltpu.VMEM((_GB,), jnp.int32),
        pltpu.VMEM((_GB,), jnp.int32),
        pltpu.VMEM((_GB, _D), jnp.float32),
        pltpu.VMEM((_SB * _D,), jnp.float32),
        pltpu.VMEM((_SB,), jnp.float32),
        pltpu.SemaphoreType.DMA,
    ],
    compiler_params=pltpu.CompilerParams(needs_layout_passes=False),
)


def _tc_body(act, x_ref, a_ref, c_ref, root_ref, w_ref, b_ref, o_ref):
    inv = 1.0 / jnp.maximum(c_ref[...], 1.0)
    acc = jnp.dot(x_ref[...], root_ref[...],
                  preferred_element_type=jnp.float32)
    acc = acc + b_ref[...]
    for r in range(_R):
        a_r = a_ref[:, r * _D:(r + 1) * _D] * inv[:, r:r + 1]
        acc = acc + jnp.dot(a_r, w_ref[r],
                            preferred_element_type=jnp.float32)
    o_ref[...] = act(acc)


def _tc_layer(x, agg, cnt128, root, w, bias, act, dout):
    return pl.pallas_call(
        functools.partial(_tc_body, act),
        grid=(_NP // _BN,),
        in_specs=[
            pl.BlockSpec((_BN, _D), lambda i: (i, 0)),
            pl.BlockSpec((_BN, _R * _D), lambda i: (i, 0)),
            pl.BlockSpec((_BN, 128), lambda i: (i, 0)),
            pl.BlockSpec((_D, dout), lambda i: (0, 0)),
            pl.BlockSpec((_R, _D, dout), lambda i: (0, 0, 0)),
            pl.BlockSpec((1, dout), lambda i: (0, 0)),
        ],
        out_specs=pl.BlockSpec((_BN, dout), lambda i: (i, 0)),
        out_shape=jax.ShapeDtypeStruct((_NP, dout), jnp.float32),
    )(x, agg, cnt128, root, w, bias)


def _relu(v):
    return jnp.maximum(v, 0.0)


def _sigmoid(v):
    return 1.0 / (1.0 + jnp.exp(-v))


@jax.jit
def kernel(edge_index, edge_type, emb, W1, root1, b1, W2, root2, b2):
    src = edge_index[0]
    dst = edge_index[1]
    seg = dst * _R + edge_type
    seg_s, src_s = lax.sort((seg, src), num_keys=1)
    seg_p = jnp.concatenate(
        [seg_s, jnp.full((_EPAD - _E,), _SEG_PAD, jnp.int32)])
    src_p = jnp.concatenate(
        [src_s, jnp.zeros((_EPAD - _E,), jnp.int32)])

    bstart = jnp.searchsorted(
        seg_s,
        jnp.arange(_NBLK_TOT + 1, dtype=jnp.int32) * _SB).astype(jnp.int32)
    bt = bstart[jnp.arange(_NW)[:, None] * _NBLK
                + jnp.arange(_NBLK + 1)[None, :]]
    bt = jnp.pad(bt, ((0, 0), (0, 3)))

    zbuf = jnp.zeros((_SB * _D,), jnp.float32)
    mesh = plsc.VectorSubcoreMesh(core_axis_name="c", subcore_axis_name="s",
                                  num_cores=_NC, num_subcores=_NS)
    sc_call = _sc_segsum(mesh=mesh)

    xp = jnp.pad(emb, ((0, _NP - _N), (0, 0)))
    agg1, cnt = sc_call(xp, src_p, seg_p, bt, zbuf)
    cnt128 = jnp.pad(cnt.reshape(_NP, _R), ((0, 0), (0, 128 - _R)))
    h1 = _tc_layer(xp, agg1.reshape(_NP, _R * _D), cnt128,
                   root1, W1, b1.reshape(1, _D), _relu, _D)

    agg2, _ = sc_call(h1, src_p, seg_p, bt, zbuf)
    w2p = jnp.pad(W2, ((0, 0), (0, 0), (0, 128 - _C)))
    root2p = jnp.pad(root2, ((0, 0), (0, 128 - _C)))
    b2p = jnp.pad(b2, (0, 128 - _C)).reshape(1, 128)
    out = _tc_layer(h1, agg2.reshape(_NP, _R * _D), cnt128,
                    root2p, w2p, b2p, _sigmoid, 128)
    return out[:_N, :_C]


# speedup vs baseline: 1.2300x; 1.0346x over previous
import functools

import jax
import jax.numpy as jnp
from jax import lax
from jax.experimental import pallas as pl
from jax.experimental.pallas import tpu as pltpu
from jax.experimental.pallas import tpu_sc as plsc

_N = 10000
_E = 160000
_R = 8
_D = 256
_C = 16

_NC = 2
_NS = 16
_NW = _NC * _NS
_NP = 10240
_SEG_PAD = _NP * _R
_SPT = _SEG_PAD // _NW
_SB = 128
_NBLK = _SPT // _SB
_NBLK_TOT = _SEG_PAD // _SB
_GB = 128
_EPAD = _E + 2 * _GB

_BN = 256


def _sc_segsum_body(x_hbm, srcs_hbm, segs_hbm, bounds_hbm, zeros_hbm,
                    agg_hbm, cnt_hbm,
                    bvec, idx_v, seg_v, rows_v, stage, cstage, sem):
    wid = lax.axis_index("c") * _NS + lax.axis_index("s")
    pltpu.sync_copy(bounds_hbm.at[wid], bvec)
    lanes = lax.iota(jnp.int32, 16)
    ones = jnp.ones((16,), jnp.float32)
    lane0 = lanes == 0
    bv0 = bvec[pl.ds(0, 16)]
    bv1 = bvec[pl.ds(8, 16)]

    def sread(b):
        lo = jnp.sum(jnp.where(lanes == b, bv0, 0))
        hi = jnp.sum(jnp.where((lanes + 8 == b) & (b >= 16), bv1, 0))
        return lo + hi

    def block_body(b, _):
        seg_lo = pl.multiple_of(wid * _SPT + b * _SB, _SB)
        e_lo = sread(b)
        e_hi = sread(b + 1)
        e_start = jnp.bitwise_and(e_lo, jnp.int32(-8))
        nbatch = (e_hi - e_start + (_GB - 1)) // _GB

        dz = pltpu.async_copy(zeros_hbm, stage, sem)
        dzc = pltpu.async_copy(zeros_hbm.at[pl.ds(0, _SB)], cstage, sem)
        dz.wait()
        dzc.wait()

        def batch_body(i, _):
            e0 = pl.multiple_of(e_start + i * _GB, 8)
            di = pltpu.async_copy(srcs_hbm.at[pl.ds(e0, _GB)], idx_v, sem)
            ds = pltpu.async_copy(segs_hbm.at[pl.ds(e0, _GB)], seg_v, sem)
            di.wait()
            ds.wait()
            pltpu.async_copy(x_hbm.at[idx_v], rows_v, sem).wait()

            def grp_body(g, _):
                seg16 = seg_v[pl.ds(g * 16, 16)]
                for k in range(16):
                    seg_sc = jnp.sum(jnp.where(lanes == k, seg16, 0))
                    seg_b = jnp.full((16,), seg_sc, jnp.int32)
                    in_blk = (seg_b >= seg_lo) & (seg_b < seg_lo + _SB)
                    base = (seg_b - seg_lo) * _D + lanes
                    row = g * 16 + k
                    for j in range(_D // 16):
                        val = rows_v[row, pl.ds(j * 16, 16)]
                        plsc.addupdate_scatter(stage, [base + j * 16], val,
                                               mask=in_blk)
                    plsc.addupdate_scatter(cstage, [seg_b - seg_lo], ones,
                                           mask=in_blk & lane0)
                return 0

            return lax.fori_loop(0, 8, grp_body, 0)

        lax.fori_loop(0, nbatch, batch_body, 0)
        da = pltpu.async_copy(
            stage,
            agg_hbm.at[pl.ds(pl.multiple_of(seg_lo * _D, 8), _SB * _D)],
            sem)
        dc = pltpu.async_copy(cstage, cnt_hbm.at[pl.ds(seg_lo, _SB)], sem)
        da.wait()
        dc.wait()
        return 0

    lax.fori_loop(0, _NBLK, block_body, 0)


_sc_segsum = functools.partial(
    pl.kernel,
    _sc_segsum_body,
    out_type=(jax.ShapeDtypeStruct((_SEG_PAD * _D,), jnp.float32),
              jax.ShapeDtypeStruct((_SEG_PAD,), jnp.float32)),
    scratch_types=[
        pltpu.VMEM((24,), jnp.int32),
        pltpu.VMEM((_GB,), jnp.int32),
        pltpu.VMEM((_GB,), jnp.int32),
        pltpu.VMEM((_GB, _D), jnp.float32),
        pltpu.VMEM((_SB * _D,), jnp.float32),
        pltpu.VMEM((_SB,), jnp.float32),
        pltpu.SemaphoreType.DMA,
    ],
    compiler_params=pltpu.CompilerParams(needs_layout_passes=False),
)


def _tc_body(act, x_ref, a_ref, c_ref, root_ref, w_ref, b_ref, o_ref):
    inv = 1.0 / jnp.maximum(c_ref[...], 1.0)
    acc = jnp.dot(x_ref[...], root_ref[...],
                  preferred_element_type=jnp.float32)
    acc = acc + b_ref[...]
    for r in range(_R):
        a_r = a_ref[:, r * _D:(r + 1) * _D] * inv[:, r:r + 1]
        acc = acc + jnp.dot(a_r, w_ref[r],
                            preferred_element_type=jnp.float32)
    o_ref[...] = act(acc)


def _tc_layer(x, agg, cnt128, root, w, bias, act, dout):
    return pl.pallas_call(
        functools.partial(_tc_body, act),
        grid=(_NP // _BN,),
        in_specs=[
            pl.BlockSpec((_BN, _D), lambda i: (i, 0)),
            pl.BlockSpec((_BN, _R * _D), lambda i: (i, 0)),
            pl.BlockSpec((_BN, 128), lambda i: (i, 0)),
            pl.BlockSpec((_D, dout), lambda i: (0, 0)),
            pl.BlockSpec((_R, _D, dout), lambda i: (0, 0, 0)),
            pl.BlockSpec((1, dout), lambda i: (0, 0)),
        ],
        out_specs=pl.BlockSpec((_BN, dout), lambda i: (i, 0)),
        out_shape=jax.ShapeDtypeStruct((_NP, dout), jnp.float32),
    )(x, agg, cnt128, root, w, bias)


def _relu(v):
    return jnp.maximum(v, 0.0)


def _sigmoid(v):
    return 1.0 / (1.0 + jnp.exp(-v))


@jax.jit
def kernel(edge_index, edge_type, emb, W1, root1, b1, W2, root2, b2):
    src = edge_index[0]
    dst = edge_index[1]
    seg = dst * _R + edge_type
    seg_s, src_s = lax.sort((seg, src), num_keys=1)
    seg_p = jnp.concatenate(
        [seg_s, jnp.full((_EPAD - _E,), _SEG_PAD, jnp.int32)])
    src_p = jnp.concatenate(
        [src_s, jnp.zeros((_EPAD - _E,), jnp.int32)])

    bstart = jnp.searchsorted(
        seg_s,
        jnp.arange(_NBLK_TOT + 1, dtype=jnp.int32) * _SB).astype(jnp.int32)
    bt = bstart[jnp.arange(_NW)[:, None] * _NBLK
                + jnp.arange(_NBLK + 1)[None, :]]
    bt = jnp.pad(bt, ((0, 0), (0, 3)))

    zbuf = jnp.zeros((_SB * _D,), jnp.float32)
    mesh = plsc.VectorSubcoreMesh(core_axis_name="c", subcore_axis_name="s",
                                  num_cores=_NC, num_subcores=_NS)
    sc_call = _sc_segsum(mesh=mesh)

    xp = jnp.pad(emb, ((0, _NP - _N), (0, 0)))
    agg1, cnt = sc_call(xp, src_p, seg_p, bt, zbuf)
    cnt128 = jnp.pad(cnt.reshape(_NP, _R), ((0, 0), (0, 128 - _R)))
    h1 = _tc_layer(xp, agg1.reshape(_NP, _R * _D), cnt128,
                   root1, W1, b1.reshape(1, _D), _relu, _D)

    agg2, _ = sc_call(h1, src_p, seg_p, bt, zbuf)
    w2p = jnp.pad(W2, ((0, 0), (0, 0), (0, 128 - _C)))
    root2p = jnp.pad(root2, ((0, 0), (0, 128 - _C)))
    b2p = jnp.pad(b2, (0, 128 - _C)).reshape(1, 128)
    out = _tc_layer(h1, agg2.reshape(_NP, _R * _D), cnt128,
                    root2p, w2p, b2p, _sigmoid, 128)
    return out[:_N, :_C]
